# baseline (speedup 1.0000x reference)
import functools

import jax
import jax.numpy as jnp
from jax import lax
from jax.experimental import pallas as pl
from jax.experimental.pallas import tpu as pltpu
from jax.experimental.pallas import tpu_sc as plsc

NC = 2
NS = 16
NW = NC * NS
LANES = 16
CH = 128
G = 1


def _make_sc_propagate(t_steps, n_nodes, h, ep):
  nch = ep // CH
  nq = 4 if (nch % 8 == 0 and h > 64) else 1
  npart = nch // nq
  njj = npart // 2
  stripe = ((n_nodes + NS * 8 - 1) // (NS * 8)) * 8
  npad = NS * stripe
  mesh = plsc.VectorSubcoreMesh(core_axis_name="c", subcore_axis_name="s",
                                num_cores=NC, num_subcores=NS)

  @functools.partial(
      pl.kernel,
      out_type=jax.ShapeDtypeStruct((NC, t_steps, NS, stripe, h),
                                    jnp.float32),
      mesh=mesh,
      scratch_types=[
          pltpu.VMEM((nch, CH), jnp.int32),
          pltpu.VMEM((npart, CH), jnp.int32),
          pltpu.VMEM((CH, h), jnp.float32),
          pltpu.VMEM((CH, h), jnp.float32),
          pltpu.VMEM_SHARED((npad, h), jnp.float32),
          pltpu.SemaphoreType.DMA,
          pltpu.SemaphoreType.DMA,
      ],
      compiler_params=pltpu.CompilerParams(use_tc_tiling_on_sc=False),
  )
  def prop(table_hbm, rowf_hbm, colr_hbm, zeros_hbm, out_hbm,
           col_v, row_v, bufa, bufb, acc, sema, semb):
    c = lax.axis_index("c")
    s = lax.axis_index("s")
    g = c * NS + s
    sl = pl.ds(s * stripe, stripe)

    pltpu.sync_copy(colr_hbm.at[g], col_v)
    pltpu.sync_copy(zeros_hbm, acc.at[sl])
    plsc.subcore_barrier()

    def step(t, _):
      for q in range(nq):
        pltpu.sync_copy(rowf_hbm.at[t, g, pl.ds(q * npart, npart)], row_v)

        def chunk2(jj, _):
          j0 = 2 * jj
          jc = q * npart + j0
          pltpu.async_copy(table_hbm.at[row_v.at[j0]], bufa, sema).wait()
          pltpu.sync_copy(bufa, acc.at[col_v.at[jc]], add=True)
          pltpu.async_copy(table_hbm.at[row_v.at[j0 + 1]], bufb, semb).wait()
          pltpu.sync_copy(bufb, acc.at[col_v.at[jc + 1]], add=True)
          return 0
        lax.fori_loop(0, njj, chunk2, 0)

      plsc.subcore_barrier()
      pltpu.sync_copy(acc.at[sl], out_hbm.at[c, t, s])
      pltpu.sync_copy(zeros_hbm, acc.at[sl])
      plsc.subcore_barrier()
      return 0
    lax.fori_loop(0, t_steps, step, 0)

  return prop


def _dis_from_deg(degh):
  return lax.rsqrt(degh[0, :, 0] + degh[1, :, 0] + 1.0)


def _gru1_body(x_ref, degh_ref, wih_ref, whh_ref, bih_ref, bhh_ref,
               out_ref, gi_ref, *, t_steps, nb, h):
  dis = _dis_from_deg(degh_ref[...])[:, None]
  xb = x_ref[...]
  d = xb.shape[-1]
  gi = jnp.dot(xb.reshape(t_steps * nb, d), wih_ref[...],
               preferred_element_type=jnp.float32) + bih_ref[...]
  gi_ref[...] = gi.reshape(t_steps, nb, 3 * h)

  hprev = jnp.zeros((nb, h), jnp.float32)
  for t in range(t_steps):
    git = gi_ref[t]
    gh = jnp.dot(hprev, whh_ref[...],
                 preferred_element_type=jnp.float32) + bhh_ref[...]
    r = jax.nn.sigmoid(git[:, :h] + gh[:, :h])
    z = jax.nn.sigmoid(git[:, h:2 * h] + gh[:, h:2 * h])
    n = jnp.tanh(git[:, 2 * h:] + r * gh[:, 2 * h:])
    hprev = (1.0 - z) * n + z * hprev
    out_ref[t // G, :, (t % G) * h:(t % G + 1) * h] = hprev * dis


def _gru2_body(xp1_ref, s1_ref, degh_ref, b1_ref, wih_ref, whh_ref, bih_ref,
               bhh_ref, out_ref, *, t_steps, nb, h_in, h):
  dis = _dis_from_deg(degh_ref[...])[:, None]
  hprev = jnp.zeros((nb, h), jnp.float32)
  for t in range(t_steps):
    g, o = t // G, (t % G) * h_in
    sall = (xp1_ref[g, :, o:o + h_in] + s1_ref[0, g, :, o:o + h_in]
            + s1_ref[1, g, :, o:o + h_in])
    xt = jax.nn.relu(sall * dis + b1_ref[...])
    git = jnp.dot(xt, wih_ref[...],
                  preferred_element_type=jnp.float32) + bih_ref[...]
    gh = jnp.dot(hprev, whh_ref[...],
                 preferred_element_type=jnp.float32) + bhh_ref[...]
    r = jax.nn.sigmoid(git[:, :h] + gh[:, :h])
    z = jax.nn.sigmoid(git[:, h:2 * h] + gh[:, h:2 * h])
    n = jnp.tanh(git[:, 2 * h:] + r * gh[:, 2 * h:])
    hprev = (1.0 - z) * n + z * hprev
  out_ref[...] = hprev * dis


def _final_body(xp2_ref, s2_ref, degh_ref, b2_ref, wl_ref, blin_ref, out_ref,
                acc_ref, *, nblocks, out_dim):
  i = pl.program_id(0)
  dis = _dis_from_deg(degh_ref[...])[:, None]
  v = (xp2_ref[...] + s2_ref[0] + s2_ref[1]) * dis + b2_ref[...]
  part = jnp.sum(wl_ref[...] * v[None], axis=(1, 2))

  @pl.when(i == 0)
  def _():
    acc_ref[0, :] = jnp.zeros((128,), jnp.float32)
  acc_ref[0, :out_dim] += part

  @pl.when(i == nblocks - 1)
  def _():
    logits = acc_ref[0, :out_dim] + blin_ref[0]
    m = jnp.max(logits)
    p = jnp.exp(logits - m)
    out_ref[0, :] = p / jnp.sum(p)


def kernel(x, edge_index, W_ih1, W_hh1, b_ih1, b_hh1, bias1,
           W_ih2, W_hh2, b_ih2, b_hh2, bias2, W_lin, b_lin):
  t_steps, n, d = x.shape
  h1 = W_hh1.shape[1]
  h2 = W_hh2.shape[1]
  out_dim = W_lin.shape[0]
  e = edge_index.shape[1]
  ep = e // NW
  tg = t_steps // G
  hg = G * h1

  f32 = jnp.float32
  i32 = jnp.int32
  stripe = ((n + NS * 8 - 1) // (NS * 8)) * 8
  npad = NS * stripe

  ep_pad = ((ep + 2 * CH - 1) // (2 * CH)) * (2 * CH)
  pad = ep_pad - ep
  nch = ep_pad // CH
  rowp = edge_index[0].reshape(NW, ep)
  colp = edge_index[1].reshape(NW, ep)
  if pad:
    pad_cols = n + (jnp.arange(pad, dtype=i32) % (npad - n))
    colp = jnp.concatenate(
        [colp, jnp.broadcast_to(pad_cols[None], (NW, pad))], axis=1)
    rowp = jnp.concatenate([rowp, jnp.zeros((NW, pad), i32)], axis=1)
  colr = colp.reshape(NW, nch, CH)
  goff = (jnp.arange(tg, dtype=i32) * n)[:, None, None]
  rowf = (rowp[None] + goff).reshape(tg, NW, nch, CH)
  row1 = rowp.reshape(1, NW, nch, CH)

  deg_prop = _make_sc_propagate(1, n, LANES, ep_pad)
  degh4 = deg_prop(jnp.ones((n, LANES), f32), row1, colr,
                   jnp.zeros((stripe, LANES), f32))
  degh = degh4.reshape(NC, npad, LANES)

  nb = 400
  grid1 = n // nb
  xp1 = pl.pallas_call(
      functools.partial(_gru1_body, t_steps=t_steps, nb=nb, h=h1),
      grid=(grid1,),
      in_specs=[
          pl.BlockSpec((t_steps, nb, d), lambda i: (0, i, 0)),
          pl.BlockSpec((NC, nb, LANES), lambda i: (0, i, 0)),
          pl.BlockSpec((d, 3 * h1), lambda i: (0, 0)),
          pl.BlockSpec((h1, 3 * h1), lambda i: (0, 0)),
          pl.BlockSpec((1, 3 * h1), lambda i: (0, 0)),
          pl.BlockSpec((1, 3 * h1), lambda i: (0, 0)),
      ],
      out_specs=pl.BlockSpec((tg, nb, hg), lambda i: (0, i, 0)),
      out_shape=jax.ShapeDtypeStruct((tg, n, hg), f32),
      scratch_shapes=[pltpu.VMEM((t_steps, nb, 3 * h1), f32)],
  )(x, degh, W_ih1.T, W_hh1.T, b_ih1.reshape(1, -1), b_hh1.reshape(1, -1))

  prop1 = _make_sc_propagate(tg, n, hg, ep_pad)
  s1 = prop1(xp1.reshape(tg * n, hg), rowf, colr,
             jnp.zeros((stripe, hg), f32))
  s1 = s1.reshape(NC, tg, npad, hg)

  xp2 = pl.pallas_call(
      functools.partial(_gru2_body, t_steps=t_steps, nb=nb, h_in=h1, h=h2),
      grid=(grid1,),
      in_specs=[
          pl.BlockSpec((tg, nb, hg), lambda i: (0, i, 0)),
          pl.BlockSpec((NC, tg, nb, hg), lambda i: (0, 0, i, 0)),
          pl.BlockSpec((NC, nb, LANES), lambda i: (0, i, 0)),
          pl.BlockSpec((1, h1), lambda i: (0, 0)),
          pl.BlockSpec((h1, 3 * h2), lambda i: (0, 0)),
          pl.BlockSpec((h2, 3 * h2), lambda i: (0, 0)),
          pl.BlockSpec((1, 3 * h2), lambda i: (0, 0)),
          pl.BlockSpec((1, 3 * h2), lambda i: (0, 0)),
      ],
      out_specs=pl.BlockSpec((nb, h2), lambda i: (i, 0)),
      out_shape=jax.ShapeDtypeStruct((n, h2), f32),
  )(xp1, s1, degh, bias1.reshape(1, -1), W_ih2.T, W_hh2.T,
    b_ih2.reshape(1, -1), b_hh2.reshape(1, -1))

  prop2 = _make_sc_propagate(1, n, h2, ep_pad)
  s2 = prop2(xp2, row1, colr,
             jnp.zeros((stripe, h2), f32)).reshape(NC, npad, h2)

  nb2 = 400
  grid3 = n // nb2
  probs = pl.pallas_call(
      functools.partial(_final_body, nblocks=grid3, out_dim=out_dim),
      grid=(grid3,),
      in_specs=[
          pl.BlockSpec((nb2, h2), lambda i: (i, 0)),
          pl.BlockSpec((NC, nb2, h2), lambda i: (0, i, 0)),
          pl.BlockSpec((NC, nb2, LANES), lambda i: (0, i, 0)),
          pl.BlockSpec((1, h2), lambda i: (0, 0)),
          pl.BlockSpec((out_dim, nb2, h2), lambda i: (0, i, 0)),
          pl.BlockSpec((1, out_dim), lambda i: (0, 0)),
      ],
      out_specs=pl.BlockSpec((1, out_dim), lambda i: (0, 0)),
      out_shape=jax.ShapeDtypeStruct((1, out_dim), f32),
      scratch_shapes=[pltpu.VMEM((8, 128), f32)],
  )(xp2, s2, degh, bias2.reshape(1, -1), W_lin.reshape(out_dim, n, h2),
    b_lin.reshape(1, -1))
  return probs


# speedup vs baseline: 1.2148x; 1.2148x over previous
import functools

import jax
import jax.numpy as jnp
from jax import lax
from jax.experimental import pallas as pl
from jax.experimental.pallas import tpu as pltpu
from jax.experimental.pallas import tpu_sc as plsc

NC = 2
NS = 16
NW = NC * NS
LANES = 16
CH = 128
G = 1


def _make_sc_propagate(t_steps, n_nodes, h, ep):
  nch = ep // CH
  nq = 4 if (nch % 8 == 0 and h > 64) else 1
  npart = nch // nq
  njj = npart // 2
  stripe = ((n_nodes + NS * 8 - 1) // (NS * 8)) * 8
  npad = NS * stripe
  mesh = plsc.VectorSubcoreMesh(core_axis_name="c", subcore_axis_name="s",
                                num_cores=NC, num_subcores=NS)

  @functools.partial(
      pl.kernel,
      out_type=jax.ShapeDtypeStruct((NC, t_steps, NS, stripe, h),
                                    jnp.float32),
      mesh=mesh,
      scratch_types=[
          pltpu.VMEM((nch, CH), jnp.int32),
          pltpu.VMEM((npart, CH), jnp.int32),
          pltpu.VMEM((CH, h), jnp.float32),
          pltpu.VMEM((CH, h), jnp.float32),
          pltpu.VMEM((stripe, h), jnp.float32),
          pltpu.VMEM_SHARED((npad, h), jnp.float32),
          pltpu.SemaphoreType.DMA,
          pltpu.SemaphoreType.DMA,
      ],
      compiler_params=pltpu.CompilerParams(use_tc_tiling_on_sc=False),
  )
  def prop(table_hbm, rowf_hbm, colr_hbm, out_hbm,
           col_v, row_v, bufa, bufb, zer_v, acc, sema, semb):
    c = lax.axis_index("c")
    s = lax.axis_index("s")
    g = c * NS + s
    sl = pl.ds(s * stripe, stripe)

    def fill_zero(i, _):
      for k in range(h // LANES):
        zer_v[i, pl.ds(k * LANES, LANES)] = jnp.zeros((LANES,), jnp.float32)
      return 0
    lax.fori_loop(0, stripe, fill_zero, 0)

    pltpu.sync_copy(colr_hbm.at[g], col_v)
    pltpu.sync_copy(zer_v, acc.at[sl])
    plsc.subcore_barrier()

    def step(t, _):
      for q in range(nq):
        pltpu.sync_copy(rowf_hbm.at[t, g, pl.ds(q * npart, npart)], row_v)
        pltpu.async_copy(table_hbm.at[row_v.at[0]], bufa, sema)

        def chunk2(jj, _):
          j0 = 2 * jj
          jc = q * npart + j0
          pltpu.async_copy(table_hbm.at[row_v.at[j0 + 1]], bufb, semb)
          pltpu.make_async_copy(table_hbm.at[row_v.at[j0]], bufa,
                                sema).wait()
          pltpu.sync_copy(bufa, acc.at[col_v.at[jc]], add=True)

          @pl.when(jj < njj - 1)
          def _():
            pltpu.async_copy(table_hbm.at[row_v.at[j0 + 2]], bufa, sema)
          pltpu.make_async_copy(table_hbm.at[row_v.at[j0 + 1]], bufb,
                                semb).wait()
          pltpu.sync_copy(bufb, acc.at[col_v.at[jc + 1]], add=True)
          return 0
        lax.fori_loop(0, njj, chunk2, 0)

      plsc.subcore_barrier()
      pltpu.sync_copy(acc.at[sl], out_hbm.at[c, t, s])
      pltpu.sync_copy(zer_v, acc.at[sl])
      plsc.subcore_barrier()
      return 0
    lax.fori_loop(0, t_steps, step, 0)

  return prop


def _dis_from_deg(degh):
  return lax.rsqrt(degh[0, :, 0] + degh[1, :, 0] + 1.0)


def _gru1_body(x_ref, degh_ref, wih_ref, whh_ref, bih_ref, bhh_ref,
               out_ref, gi_ref, *, t_steps, nb, h):
  dis = _dis_from_deg(degh_ref[...])[:, None]
  xb = x_ref[...]
  d = xb.shape[-1]
  gi = jnp.dot(xb.reshape(t_steps * nb, d), wih_ref[...],
               preferred_element_type=jnp.float32) + bih_ref[...]
  gi_ref[...] = gi.reshape(t_steps, nb, 3 * h)

  hprev = jnp.zeros((nb, h), jnp.float32)
  for t in range(t_steps):
    git = gi_ref[t]
    gh = jnp.dot(hprev, whh_ref[...],
                 preferred_element_type=jnp.float32) + bhh_ref[...]
    r = jax.nn.sigmoid(git[:, :h] + gh[:, :h])
    z = jax.nn.sigmoid(git[:, h:2 * h] + gh[:, h:2 * h])
    n = jnp.tanh(git[:, 2 * h:] + r * gh[:, 2 * h:])
    hprev = (1.0 - z) * n + z * hprev
    out_ref[t // G, :, (t % G) * h:(t % G + 1) * h] = hprev * dis


def _gru2_body(xp1_ref, s1_ref, degh_ref, b1_ref, wih_ref, whh_ref, bih_ref,
               bhh_ref, out_ref, *, t_steps, nb, h_in, h):
  dis = _dis_from_deg(degh_ref[...])[:, None]
  hprev = jnp.zeros((nb, h), jnp.float32)
  for t in range(t_steps):
    g, o = t // G, (t % G) * h_in
    sall = (xp1_ref[g, :, o:o + h_in] + s1_ref[0, g, :, o:o + h_in]
            + s1_ref[1, g, :, o:o + h_in])
    xt = jax.nn.relu(sall * dis + b1_ref[...])
    git = jnp.dot(xt, wih_ref[...],
                  preferred_element_type=jnp.float32) + bih_ref[...]
    gh = jnp.dot(hprev, whh_ref[...],
                 preferred_element_type=jnp.float32) + bhh_ref[...]
    r = jax.nn.sigmoid(git[:, :h] + gh[:, :h])
    z = jax.nn.sigmoid(git[:, h:2 * h] + gh[:, h:2 * h])
    n = jnp.tanh(git[:, 2 * h:] + r * gh[:, 2 * h:])
    hprev = (1.0 - z) * n + z * hprev
  out_ref[...] = hprev * dis


def _final_body(xp2_ref, s2_ref, degh_ref, b2_ref, wl_ref, blin_ref, out_ref,
                acc_ref, *, nblocks, out_dim):
  i = pl.program_id(0)
  dis = _dis_from_deg(degh_ref[...])[:, None]
  v = (xp2_ref[...] + s2_ref[0] + s2_ref[1]) * dis + b2_ref[...]
  part = jnp.sum(wl_ref[...] * v[None], axis=(1, 2))

  @pl.when(i == 0)
  def _():
    acc_ref[0, :] = jnp.zeros((128,), jnp.float32)
  acc_ref[0, :out_dim] += part

  @pl.when(i == nblocks - 1)
  def _():
    logits = acc_ref[0, :out_dim] + blin_ref[0]
    m = jnp.max(logits)
    p = jnp.exp(logits - m)
    out_ref[0, :] = p / jnp.sum(p)


def kernel(x, edge_index, W_ih1, W_hh1, b_ih1, b_hh1, bias1,
           W_ih2, W_hh2, b_ih2, b_hh2, bias2, W_lin, b_lin):
  t_steps, n, d = x.shape
  h1 = W_hh1.shape[1]
  h2 = W_hh2.shape[1]
  out_dim = W_lin.shape[0]
  e = edge_index.shape[1]
  ep = e // NW
  tg = t_steps // G
  hg = G * h1

  f32 = jnp.float32
  i32 = jnp.int32
  stripe = ((n + NS * 8 - 1) // (NS * 8)) * 8
  npad = NS * stripe

  ep_pad = ((ep + 2 * CH - 1) // (2 * CH)) * (2 * CH)
  pad = ep_pad - ep
  nch = ep_pad // CH
  rowp = edge_index[0].reshape(NW, ep)
  colp = edge_index[1].reshape(NW, ep)
  if pad:
    pad_cols = n + (jnp.arange(pad, dtype=i32) % (npad - n))
    colp = jnp.concatenate(
        [colp, jnp.broadcast_to(pad_cols[None], (NW, pad))], axis=1)
    rowp = jnp.concatenate([rowp, jnp.zeros((NW, pad), i32)], axis=1)
  colr = colp.reshape(NW, nch, CH)
  goff = (jnp.arange(tg, dtype=i32) * n)[:, None, None]
  rowf = (rowp[None] + goff).reshape(tg, NW, nch, CH)
  row1 = rowp.reshape(1, NW, nch, CH)

  deg_prop = _make_sc_propagate(1, n, LANES, ep_pad)
  degh4 = deg_prop(jnp.ones((n, LANES), f32), row1, colr)
  degh = degh4.reshape(NC, npad, LANES)

  nb = 400
  grid1 = n // nb
  xp1 = pl.pallas_call(
      functools.partial(_gru1_body, t_steps=t_steps, nb=nb, h=h1),
      grid=(grid1,),
      in_specs=[
          pl.BlockSpec((t_steps, nb, d), lambda i: (0, i, 0)),
          pl.BlockSpec((NC, nb, LANES), lambda i: (0, i, 0)),
          pl.BlockSpec((d, 3 * h1), lambda i: (0, 0)),
          pl.BlockSpec((h1, 3 * h1), lambda i: (0, 0)),
          pl.BlockSpec((1, 3 * h1), lambda i: (0, 0)),
          pl.BlockSpec((1, 3 * h1), lambda i: (0, 0)),
      ],
      out_specs=pl.BlockSpec((tg, nb, hg), lambda i: (0, i, 0)),
      out_shape=jax.ShapeDtypeStruct((tg, n, hg), f32),
      scratch_shapes=[pltpu.VMEM((t_steps, nb, 3 * h1), f32)],
  )(x, degh, W_ih1.T, W_hh1.T, b_ih1.reshape(1, -1), b_hh1.reshape(1, -1))

  prop1 = _make_sc_propagate(tg, n, hg, ep_pad)
  s1 = prop1(xp1.reshape(tg * n, hg), rowf, colr)
  s1 = s1.reshape(NC, tg, npad, hg)

  xp2 = pl.pallas_call(
      functools.partial(_gru2_body, t_steps=t_steps, nb=nb, h_in=h1, h=h2),
      grid=(grid1,),
      in_specs=[
          pl.BlockSpec((tg, nb, hg), lambda i: (0, i, 0)),
          pl.BlockSpec((NC, tg, nb, hg), lambda i: (0, 0, i, 0)),
          pl.BlockSpec((NC, nb, LANES), lambda i: (0, i, 0)),
          pl.BlockSpec((1, h1), lambda i: (0, 0)),
          pl.BlockSpec((h1, 3 * h2), lambda i: (0, 0)),
          pl.BlockSpec((h2, 3 * h2), lambda i: (0, 0)),
          pl.BlockSpec((1, 3 * h2), lambda i: (0, 0)),
          pl.BlockSpec((1, 3 * h2), lambda i: (0, 0)),
      ],
      out_specs=pl.BlockSpec((nb, h2), lambda i: (i, 0)),
      out_shape=jax.ShapeDtypeStruct((n, h2), f32),
  )(xp1, s1, degh, bias1.reshape(1, -1), W_ih2.T, W_hh2.T,
    b_ih2.reshape(1, -1), b_hh2.reshape(1, -1))

  prop2 = _make_sc_propagate(1, n, h2, ep_pad)
  s2 = prop2(xp2, row1, colr).reshape(NC, npad, h2)

  nb2 = 400
  grid3 = n // nb2
  probs = pl.pallas_call(
      functools.partial(_final_body, nblocks=grid3, out_dim=out_dim),
      grid=(grid3,),
      in_specs=[
          pl.BlockSpec((nb2, h2), lambda i: (i, 0)),
          pl.BlockSpec((NC, nb2, h2), lambda i: (0, i, 0)),
          pl.BlockSpec((NC, nb2, LANES), lambda i: (0, i, 0)),
          pl.BlockSpec((1, h2), lambda i: (0, 0)),
          pl.BlockSpec((out_dim, nb2, h2), lambda i: (0, i, 0)),
          pl.BlockSpec((1, out_dim), lambda i: (0, 0)),
      ],
      out_specs=pl.BlockSpec((1, out_dim), lambda i: (0, 0)),
      out_shape=jax.ShapeDtypeStruct((1, out_dim), f32),
      scratch_shapes=[pltpu.VMEM((8, 128), f32)],
  )(xp2, s2, degh, bias2.reshape(1, -1), W_lin.reshape(out_dim, n, h2),
    b_lin.reshape(1, -1))
  return probs


# speedup vs baseline: 2.6523x; 2.1833x over previous
import functools

import jax
import jax.numpy as jnp
from jax import lax
from jax.experimental import pallas as pl
from jax.experimental.pallas import tpu as pltpu
from jax.experimental.pallas import tpu_sc as plsc

NC = 2
NS = 16
NW = NC * NS
LANES = 16
CH = 100
G = 1


def _make_sc_propagate(t_steps, n_nodes, h, ep):
  nch = ep // CH
  nq = 4 if (nch % 8 == 0 and h > 64) else 1
  npart = nch // nq
  njj = npart // 2
  stripe = ((n_nodes + NS * 8 - 1) // (NS * 8)) * 8
  npad = NS * stripe
  mesh = plsc.VectorSubcoreMesh(core_axis_name="c", subcore_axis_name="s",
                                num_cores=NC, num_subcores=NS)

  @functools.partial(
      pl.kernel,
      out_type=jax.ShapeDtypeStruct((NC, t_steps, NS, stripe, h),
                                    jnp.float32),
      mesh=mesh,
      scratch_types=[
          pltpu.VMEM((nch, CH), jnp.int32),
          pltpu.VMEM((npart, CH), jnp.int32),
          pltpu.VMEM((CH, h), jnp.float32),
          pltpu.VMEM((CH, h), jnp.float32),
          pltpu.VMEM((stripe, h), jnp.float32),
          pltpu.VMEM_SHARED((npad, h), jnp.float32),
          pltpu.SemaphoreType.DMA,
          pltpu.SemaphoreType.DMA,
      ],
      compiler_params=pltpu.CompilerParams(use_tc_tiling_on_sc=False),
  )
  def prop(table_hbm, rowf_hbm, colr_hbm, out_hbm,
           col_v, row_v, bufa, bufb, zer_v, acc, sema, semb):
    c = lax.axis_index("c")
    s = lax.axis_index("s")
    g = c * NS + s
    sl = pl.ds(s * stripe, stripe)

    def fill_zero(i, _):
      for k in range(h // LANES):
        zer_v[i, pl.ds(k * LANES, LANES)] = jnp.zeros((LANES,), jnp.float32)
      return 0
    lax.fori_loop(0, stripe, fill_zero, 0)

    pltpu.sync_copy(colr_hbm.at[g], col_v)
    pltpu.sync_copy(zer_v, acc.at[sl])
    plsc.subcore_barrier()

    def step(t, _):
      for q in range(nq):
        pltpu.sync_copy(rowf_hbm.at[t, g, pl.ds(q * npart, npart)], row_v)
        pltpu.async_copy(table_hbm.at[row_v.at[0]], bufa, sema)

        def chunk2(jj, _):
          j0 = 2 * jj
          jc = q * npart + j0
          pltpu.async_copy(table_hbm.at[row_v.at[j0 + 1]], bufb, semb)
          pltpu.make_async_copy(table_hbm.at[row_v.at[j0]], bufa,
                                sema).wait()
          pltpu.sync_copy(bufa, acc.at[col_v.at[jc]], add=True)

          @pl.when(jj < njj - 1)
          def _():
            pltpu.async_copy(table_hbm.at[row_v.at[j0 + 2]], bufa, sema)
          pltpu.make_async_copy(table_hbm.at[row_v.at[j0 + 1]], bufb,
                                semb).wait()
          pltpu.sync_copy(bufb, acc.at[col_v.at[jc + 1]], add=True)
          return 0
        lax.fori_loop(0, njj, chunk2, 0)

      plsc.subcore_barrier()
      pltpu.sync_copy(acc.at[sl], out_hbm.at[c, t, s])
      pltpu.sync_copy(zer_v, acc.at[sl])
      plsc.subcore_barrier()
      return 0
    lax.fori_loop(0, t_steps, step, 0)

  return prop


def _dis_from_deg(degh):
  return lax.rsqrt(degh[0, :, 0] + degh[1, :, 0] + 1.0)


def _gru1_body(x_ref, degh_ref, wih_ref, whh_ref, bih_ref, bhh_ref,
               out_ref, gi_ref, *, t_steps, nb, h):
  dis = _dis_from_deg(degh_ref[...])[:, None]
  xb = x_ref[...]
  d = xb.shape[-1]
  gi = jnp.dot(xb.reshape(t_steps * nb, d), wih_ref[...],
               preferred_element_type=jnp.float32) + bih_ref[...]
  gi_ref[...] = gi.reshape(t_steps, nb, 3 * h)

  hprev = jnp.zeros((nb, h), jnp.float32)
  for t in range(t_steps):
    git = gi_ref[t]
    gh = jnp.dot(hprev, whh_ref[...],
                 preferred_element_type=jnp.float32) + bhh_ref[...]
    r = jax.nn.sigmoid(git[:, :h] + gh[:, :h])
    z = jax.nn.sigmoid(git[:, h:2 * h] + gh[:, h:2 * h])
    n = jnp.tanh(git[:, 2 * h:] + r * gh[:, 2 * h:])
    hprev = (1.0 - z) * n + z * hprev
    out_ref[t // G, :, (t % G) * h:(t % G + 1) * h] = hprev * dis


def _gru2_body(xp1_ref, s1_ref, degh_ref, b1_ref, wih_ref, whh_ref, bih_ref,
               bhh_ref, out_ref, *, t_steps, nb, h_in, h):
  dis = _dis_from_deg(degh_ref[...])[:, None]
  hprev = jnp.zeros((nb, h), jnp.float32)
  for t in range(t_steps):
    g, o = t // G, (t % G) * h_in
    sall = (xp1_ref[g, :, o:o + h_in] + s1_ref[0, g, :, o:o + h_in]
            + s1_ref[1, g, :, o:o + h_in])
    xt = jax.nn.relu(sall * dis + b1_ref[...])
    git = jnp.dot(xt, wih_ref[...],
                  preferred_element_type=jnp.float32) + bih_ref[...]
    gh = jnp.dot(hprev, whh_ref[...],
                 preferred_element_type=jnp.float32) + bhh_ref[...]
    r = jax.nn.sigmoid(git[:, :h] + gh[:, :h])
    z = jax.nn.sigmoid(git[:, h:2 * h] + gh[:, h:2 * h])
    n = jnp.tanh(git[:, 2 * h:] + r * gh[:, 2 * h:])
    hprev = (1.0 - z) * n + z * hprev
  out_ref[...] = hprev * dis


def _final_body(xp2_ref, s2_ref, degh_ref, b2_ref, wl_ref, blin_ref, out_ref,
                acc_ref, *, nblocks, out_dim):
  i = pl.program_id(0)
  dis = _dis_from_deg(degh_ref[...])[:, None]
  v = (xp2_ref[...] + s2_ref[0] + s2_ref[1]) * dis + b2_ref[...]
  part = jnp.sum(wl_ref[...] * v[None], axis=(1, 2))

  @pl.when(i == 0)
  def _():
    acc_ref[0, :] = jnp.zeros((128,), jnp.float32)
  acc_ref[0, :out_dim] += part

  @pl.when(i == nblocks - 1)
  def _():
    logits = acc_ref[0, :out_dim] + blin_ref[0]
    m = jnp.max(logits)
    p = jnp.exp(logits - m)
    out_ref[0, :] = p / jnp.sum(p)


def kernel(x, edge_index, W_ih1, W_hh1, b_ih1, b_hh1, bias1,
           W_ih2, W_hh2, b_ih2, b_hh2, bias2, W_lin, b_lin):
  t_steps, n, d = x.shape
  h1 = W_hh1.shape[1]
  h2 = W_hh2.shape[1]
  out_dim = W_lin.shape[0]
  e = edge_index.shape[1]
  ep = e // NW
  tg = t_steps // G
  hg = G * h1

  f32 = jnp.float32
  i32 = jnp.int32
  stripe = ((n + NS * 8 - 1) // (NS * 8)) * 8
  npad = NS * stripe

  ep_pad = ((ep + 2 * CH - 1) // (2 * CH)) * (2 * CH)
  pad = ep_pad - ep
  nch = ep_pad // CH
  rowp = edge_index[0].reshape(NW, ep)
  colp = edge_index[1].reshape(NW, ep)
  if pad:
    pad_cols = n + (jnp.arange(pad, dtype=i32) % (npad - n))
    colp = jnp.concatenate(
        [colp, jnp.broadcast_to(pad_cols[None], (NW, pad))], axis=1)
    rowp = jnp.concatenate([rowp, jnp.zeros((NW, pad), i32)], axis=1)
  colr = colp.reshape(NW, nch, CH)
  goff = (jnp.arange(tg, dtype=i32) * n)[:, None, None]
  rowf = (rowp[None] + goff).reshape(tg, NW, nch, CH)
  row1 = rowp.reshape(1, NW, nch, CH)

  deg_prop = _make_sc_propagate(1, n, LANES, ep_pad)
  degh4 = deg_prop(jnp.ones((n, LANES), f32), row1, colr)
  degh = degh4.reshape(NC, npad, LANES)

  nb = 400
  grid1 = n // nb
  xp1 = pl.pallas_call(
      functools.partial(_gru1_body, t_steps=t_steps, nb=nb, h=h1),
      grid=(grid1,),
      in_specs=[
          pl.BlockSpec((t_steps, nb, d), lambda i: (0, i, 0)),
          pl.BlockSpec((NC, nb, LANES), lambda i: (0, i, 0)),
          pl.BlockSpec((d, 3 * h1), lambda i: (0, 0)),
          pl.BlockSpec((h1, 3 * h1), lambda i: (0, 0)),
          pl.BlockSpec((1, 3 * h1), lambda i: (0, 0)),
          pl.BlockSpec((1, 3 * h1), lambda i: (0, 0)),
      ],
      out_specs=pl.BlockSpec((tg, nb, hg), lambda i: (0, i, 0)),
      out_shape=jax.ShapeDtypeStruct((tg, n, hg), f32),
      scratch_shapes=[pltpu.VMEM((t_steps, nb, 3 * h1), f32)],
  )(x, degh, W_ih1.T, W_hh1.T, b_ih1.reshape(1, -1), b_hh1.reshape(1, -1))

  prop1 = _make_sc_propagate(tg, n, hg, ep_pad)
  s1 = prop1(xp1.reshape(tg * n, hg), rowf, colr)
  s1 = s1.reshape(NC, tg, npad, hg)

  xp2 = pl.pallas_call(
      functools.partial(_gru2_body, t_steps=t_steps, nb=nb, h_in=h1, h=h2),
      grid=(grid1,),
      in_specs=[
          pl.BlockSpec((tg, nb, hg), lambda i: (0, i, 0)),
          pl.BlockSpec((NC, tg, nb, hg), lambda i: (0, 0, i, 0)),
          pl.BlockSpec((NC, nb, LANES), lambda i: (0, i, 0)),
          pl.BlockSpec((1, h1), lambda i: (0, 0)),
          pl.BlockSpec((h1, 3 * h2), lambda i: (0, 0)),
          pl.BlockSpec((h2, 3 * h2), lambda i: (0, 0)),
          pl.BlockSpec((1, 3 * h2), lambda i: (0, 0)),
          pl.BlockSpec((1, 3 * h2), lambda i: (0, 0)),
      ],
      out_specs=pl.BlockSpec((nb, h2), lambda i: (i, 0)),
      out_shape=jax.ShapeDtypeStruct((n, h2), f32),
  )(xp1, s1, degh, bias1.reshape(1, -1), W_ih2.T, W_hh2.T,
    b_ih2.reshape(1, -1), b_hh2.reshape(1, -1))

  prop2 = _make_sc_propagate(1, n, h2, ep_pad)
  s2 = prop2(xp2, row1, colr).reshape(NC, npad, h2)

  nb2 = 400
  grid3 = n // nb2
  probs = pl.pallas_call(
      functools.partial(_final_body, nblocks=grid3, out_dim=out_dim),
      grid=(grid3,),
      in_specs=[
          pl.BlockSpec((nb2, h2), lambda i: (i, 0)),
          pl.BlockSpec((NC, nb2, h2), lambda i: (0, i, 0)),
          pl.BlockSpec((NC, nb2, LANES), lambda i: (0, i, 0)),
          pl.BlockSpec((1, h2), lambda i: (0, 0)),
          pl.BlockSpec((out_dim, nb2, h2), lambda i: (0, i, 0)),
          pl.BlockSpec((1, out_dim), lambda i: (0, 0)),
      ],
      out_specs=pl.BlockSpec((1, out_dim), lambda i: (0, 0)),
      out_shape=jax.ShapeDtypeStruct((1, out_dim), f32),
      scratch_shapes=[pltpu.VMEM((8, 128), f32)],
  )(xp2, s2, degh, bias2.reshape(1, -1), W_lin.reshape(out_dim, n, h2),
    b_lin.reshape(1, -1))
  return probs
